# baseline (speedup 1.0000x reference)
import functools

import jax
import jax.numpy as jnp
from jax import lax
from jax.experimental import pallas as pl
from jax.experimental.pallas import tpu as pltpu
from jax.experimental.pallas import tpu_sc as plsc

_SC_CORES = 2
_SC_SUBCORES = 16
_NW = _SC_CORES * _SC_SUBCORES
_HBITS = 12
_HBINS = 1 << _HBITS
_HSIZE = _HBINS * 16
_MIN32 = -2147483648


def _sc_hist_body(shard, chunk, nchunk, x_hbm, out_hbm, buf0, buf1, hist_v):
    cid = lax.axis_index("c")
    sid = lax.axis_index("s")
    wid = cid * _SC_SUBCORES + sid
    base = wid * shard
    lane = lax.iota(jnp.int32, 16)
    ones = jnp.ones((16,), jnp.int32)

    def zero(i, _):
        hist_v[pl.ds(i * 16, 16)] = jnp.zeros((16,), jnp.int32)
        return 0

    lax.fori_loop(0, _HSIZE // 16, zero, 0)

    def process(buf):
        def inner(v, _):
            ib = buf[pl.ds(v * 16, 16)]
            m = lax.shift_right_arithmetic(ib, 31)
            key = lax.bitwise_xor(ib, lax.bitwise_or(m, jnp.int32(_MIN32)))
            idx = lax.bitwise_or(
                lax.bitwise_and(
                    lax.shift_right_logical(key, 32 - _HBITS - 4),
                    jnp.int32((_HBINS - 1) << 4),
                ),
                lane,
            )
            plsc.addupdate_scatter(hist_v, (idx,), ones)
            return 0

        lax.fori_loop(0, chunk // 16, inner, 0)

    def loop(c2, _):
        for bi, bf in enumerate((buf0, buf1)):
            c = c2 * 2 + bi
            pltpu.sync_copy(x_hbm.at[pl.ds(base + c * chunk, chunk)], bf)
            process(bf)
        return 0

    lax.fori_loop(0, nchunk // 2, loop, 0)
    pltpu.sync_copy(hist_v, out_hbm.at[wid])


def _sc_hist(xflat, b, n, chunk, interpret=False):
    wpr = _NW // b
    shard = n // wpr
    nchunk = shard // chunk
    assert shard * wpr == n and nchunk * chunk == shard and nchunk % 2 == 0
    mesh = plsc.VectorSubcoreMesh(core_axis_name="c", subcore_axis_name="s")
    f = pl.kernel(
        functools.partial(_sc_hist_body, shard, chunk, nchunk),
        out_type=jax.ShapeDtypeStruct((_NW, _HSIZE), jnp.int32),
        mesh=mesh,
        scratch_types=[
            pltpu.VMEM((chunk,), jnp.int32),
            pltpu.VMEM((chunk,), jnp.int32),
            pltpu.VMEM((_HSIZE,), jnp.int32),
        ],
        compiler_params=pltpu.CompilerParams(needs_layout_passes=False),
        interpret=interpret,
    )
    return f(xflat)


def _kwta_row_kernel(k, x_hbm, p0_ref, out_hbm, buf, key_buf, in_sem, out_sem):
    b = pl.program_id(0)
    cp_in = pltpu.make_async_copy(x_hbm.at[b], buf, in_sem)
    cp_in.start()
    cp_in.wait()

    xv = buf[...]
    ibits = pltpu.bitcast(xv, jnp.int32)
    skey = ibits ^ jnp.bitwise_and(
        jnp.right_shift(ibits, 31), jnp.int32(0x7FFFFFFF)
    )
    key_buf[...] = skey

    rows = key_buf.shape[0]
    acc_rows = 64

    def step(i, p):
        cand = p | (jnp.int32(1) << (31 - i))
        st = cand ^ jnp.int32(-2147483648)

        def chunk(j, acc):
            blk = key_buf[pl.ds(j * acc_rows, acc_rows), :]
            return acc + (blk >= st).astype(jnp.int32)

        acc = jax.lax.fori_loop(
            0,
            rows // acc_rows,
            chunk,
            jnp.zeros((acc_rows, 128), jnp.int32),
            unroll=8,
        )
        cnt = jnp.sum(acc)
        return jnp.where(cnt >= k, cand, p)

    p_final = jax.lax.fori_loop(_HBITS, 32, step, p0_ref[b], unroll=False)
    kth_key = p_final ^ jnp.int32(-2147483648)

    buf[...] = jnp.where(key_buf[...] >= kth_key, xv, jnp.float32(0.0))

    cp_out = pltpu.make_async_copy(buf, out_hbm.at[b], out_sem)
    cp_out.start()
    cp_out.wait()


def _kwta_2d(x2d, k, chunk=12288, interpret=False):
    b, n = x2d.shape
    assert n % 1024 == 0, n
    rows = n // 128
    x3 = x2d.reshape(b, rows, 128)

    hist = _sc_hist(lax.bitcast_convert_type(x2d.reshape(b * n), jnp.int32), b, n, chunk, interpret=interpret)
    h = hist.reshape(b, _NW // b, _HBINS, 16).sum(axis=(1, 3))
    surv = jnp.cumsum(h[:, ::-1], axis=1)[:, ::-1]
    b1 = jnp.sum((surv >= k).astype(jnp.int32), axis=1) - 1
    p0 = jnp.left_shift(b1, 32 - _HBITS)

    out = pl.pallas_call(
        functools.partial(_kwta_row_kernel, k),
        grid=(b,),
        in_specs=[
            pl.BlockSpec(memory_space=pl.ANY),
            pl.BlockSpec(memory_space=pltpu.SMEM),
        ],
        out_specs=pl.BlockSpec(memory_space=pl.ANY),
        out_shape=jax.ShapeDtypeStruct((b, rows, 128), jnp.float32),
        scratch_shapes=[
            pltpu.VMEM((rows, 128), jnp.float32),
            pltpu.VMEM((rows, 128), jnp.int32),
            pltpu.SemaphoreType.DMA,
            pltpu.SemaphoreType.DMA,
        ],
        interpret=interpret,
    )(x3, p0)
    return out.reshape(b, n)


def kernel(x):
    b = x.shape[0]
    size = x.shape[1] * x.shape[2] * x.shape[3]
    k = int(0.1 * size)
    out = _kwta_2d(x.reshape(b, size), k)
    return out.reshape(x.shape)


# speedup vs baseline: 1.0477x; 1.0477x over previous
import functools

import jax
import jax.numpy as jnp
from jax import lax
from jax.experimental import pallas as pl
from jax.experimental.pallas import tpu as pltpu
from jax.experimental.pallas import tpu_sc as plsc

_SC_CORES = 2
_SC_SUBCORES = 16
_NW = _SC_CORES * _SC_SUBCORES
_HBITS = 12
_HBINS = 1 << _HBITS
_HSIZE = _HBINS * 16
_MIN32 = -2147483648


def _sc_hist_body(shard, chunk, nchunk, x_hbm, out_hbm, buf0, buf1, hist_v):
    cid = lax.axis_index("c")
    sid = lax.axis_index("s")
    wid = cid * _SC_SUBCORES + sid
    base = wid * shard
    lane = lax.iota(jnp.int32, 16)
    ones = jnp.ones((16,), jnp.int32)

    def zero(i, _):
        hist_v[pl.ds(i * 16, 16)] = jnp.zeros((16,), jnp.int32)
        return 0

    lax.fori_loop(0, _HSIZE // 16, zero, 0, unroll=8)

    def process(buf):
        def inner(v, _):
            ib = buf[pl.ds(v * 16, 16)]
            m = lax.shift_right_arithmetic(ib, 31)
            key = lax.bitwise_xor(ib, lax.bitwise_or(m, jnp.int32(_MIN32)))
            idx = lax.bitwise_or(
                lax.bitwise_and(
                    lax.shift_right_logical(key, 32 - _HBITS - 4),
                    jnp.int32((_HBINS - 1) << 4),
                ),
                lane,
            )
            plsc.addupdate_scatter(hist_v, (idx,), ones)
            return 0

        lax.fori_loop(0, chunk // 16, inner, 0, unroll=8)

    def loop(c2, _):
        for bi, bf in enumerate((buf0, buf1)):
            c = c2 * 2 + bi
            pltpu.sync_copy(x_hbm.at[pl.ds(base + c * chunk, chunk)], bf)
            process(bf)
        return 0

    lax.fori_loop(0, nchunk // 2, loop, 0)
    pltpu.sync_copy(hist_v, out_hbm.at[wid])


def _sc_hist(xflat, b, n, chunk, interpret=False):
    wpr = _NW // b
    shard = n // wpr
    nchunk = shard // chunk
    assert shard * wpr == n and nchunk * chunk == shard and nchunk % 2 == 0
    mesh = plsc.VectorSubcoreMesh(core_axis_name="c", subcore_axis_name="s")
    f = pl.kernel(
        functools.partial(_sc_hist_body, shard, chunk, nchunk),
        out_type=jax.ShapeDtypeStruct((_NW, _HSIZE), jnp.int32),
        mesh=mesh,
        scratch_types=[
            pltpu.VMEM((chunk,), jnp.int32),
            pltpu.VMEM((chunk,), jnp.int32),
            pltpu.VMEM((_HSIZE,), jnp.int32),
        ],
        compiler_params=pltpu.CompilerParams(needs_layout_passes=False),
        interpret=interpret,
    )
    return f(xflat)


def _kwta_row_kernel(k, x_hbm, p0_ref, out_hbm, buf, key_buf, in_sem, out_sem):
    b = pl.program_id(0)
    cp_in = pltpu.make_async_copy(x_hbm.at[b], buf, in_sem)
    cp_in.start()
    cp_in.wait()

    xv = buf[...]
    ibits = pltpu.bitcast(xv, jnp.int32)
    skey = ibits ^ jnp.bitwise_and(
        jnp.right_shift(ibits, 31), jnp.int32(0x7FFFFFFF)
    )
    key_buf[...] = skey

    rows = key_buf.shape[0]
    acc_rows = 64

    def step(i, p):
        cand = p | (jnp.int32(1) << (31 - i))
        st = cand ^ jnp.int32(-2147483648)

        def chunk(j, acc):
            blk = key_buf[pl.ds(j * acc_rows, acc_rows), :]
            return acc + (blk >= st).astype(jnp.int32)

        acc = jax.lax.fori_loop(
            0,
            rows // acc_rows,
            chunk,
            jnp.zeros((acc_rows, 128), jnp.int32),
            unroll=8,
        )
        cnt = jnp.sum(acc)
        return jnp.where(cnt >= k, cand, p)

    p_final = jax.lax.fori_loop(_HBITS, 32, step, p0_ref[b], unroll=False)
    kth_key = p_final ^ jnp.int32(-2147483648)

    buf[...] = jnp.where(key_buf[...] >= kth_key, xv, jnp.float32(0.0))

    cp_out = pltpu.make_async_copy(buf, out_hbm.at[b], out_sem)
    cp_out.start()
    cp_out.wait()


def _kwta_2d(x2d, k, chunk=12288, interpret=False):
    b, n = x2d.shape
    assert n % 1024 == 0, n
    rows = n // 128
    x3 = x2d.reshape(b, rows, 128)

    hist = _sc_hist(lax.bitcast_convert_type(x2d.reshape(b * n), jnp.int32), b, n, chunk, interpret=interpret)
    h = hist.reshape(b, _NW // b, _HBINS, 16).sum(axis=(1, 3))
    surv = jnp.cumsum(h[:, ::-1], axis=1)[:, ::-1]
    b1 = jnp.sum((surv >= k).astype(jnp.int32), axis=1) - 1
    p0 = jnp.left_shift(b1, 32 - _HBITS)

    out = pl.pallas_call(
        functools.partial(_kwta_row_kernel, k),
        grid=(b,),
        in_specs=[
            pl.BlockSpec(memory_space=pl.ANY),
            pl.BlockSpec(memory_space=pltpu.SMEM),
        ],
        out_specs=pl.BlockSpec(memory_space=pl.ANY),
        out_shape=jax.ShapeDtypeStruct((b, rows, 128), jnp.float32),
        scratch_shapes=[
            pltpu.VMEM((rows, 128), jnp.float32),
            pltpu.VMEM((rows, 128), jnp.int32),
            pltpu.SemaphoreType.DMA,
            pltpu.SemaphoreType.DMA,
        ],
        interpret=interpret,
    )(x3, p0)
    return out.reshape(b, n)


def kernel(x):
    b = x.shape[0]
    size = x.shape[1] * x.shape[2] * x.shape[3]
    k = int(0.1 * size)
    out = _kwta_2d(x.reshape(b, size), k)
    return out.reshape(x.shape)
